# baseline (speedup 1.0000x reference)
import functools

import jax
import jax.numpy as jnp
from jax import lax
from jax.experimental import pallas as pl
from jax.experimental.pallas import tpu as pltpu
from jax.experimental.pallas import tpu_sc as plsc

NUM_WORKERS = 32
CHUNK = 832
NBUF = 4


def _emb_gather(idx_hbm, table_hbm, out_hbm, idx_v, rows_v, gsems, wsems,
                *, b_per_w, n_chunks):
    wid = lax.axis_index("s") * 2 + lax.axis_index("c")
    base = wid * b_per_w

    pltpu.sync_copy(idx_hbm.at[pl.ds(base, b_per_w)], idx_v)

    def gather_start(ch, b):
        pltpu.async_copy(
            table_hbm.at[idx_v.at[pl.ds(ch * CHUNK, CHUNK)]],
            rows_v.at[b], gsems.at[b])

    def gather_wait(ch, b):
        pltpu.make_async_copy(
            table_hbm.at[idx_v.at[pl.ds(ch * CHUNK, CHUNK)]],
            rows_v.at[b], gsems.at[b]).wait()

    def writeback_start(ch, b):
        pltpu.async_copy(
            rows_v.at[b], out_hbm.at[pl.ds(base + ch * CHUNK, CHUNK)],
            wsems.at[b])

    def writeback_wait(ch, b):
        pltpu.make_async_copy(
            rows_v.at[b], out_hbm.at[pl.ds(base + ch * CHUNK, CHUNK)],
            wsems.at[b]).wait()

    for b in range(min(NBUF, n_chunks)):
        gather_start(b, b)
    for ch in range(n_chunks):
        b = ch % NBUF
        gather_wait(ch, b)
        writeback_start(ch, b)
        if ch + NBUF < n_chunks:
            writeback_wait(ch, b)
            gather_start(ch + NBUF, b)
    for ch in range(max(0, n_chunks - NBUF), n_chunks):
        writeback_wait(ch, ch % NBUF)


def kernel(token_ids, embeddings):
    batch, fields = token_ids.shape
    num_rows, dim = embeddings.shape
    total = batch * fields
    b_per_w = total // NUM_WORKERS
    n_chunks = b_per_w // CHUNK

    idx_flat = token_ids.reshape(total).astype(jnp.int32)

    eye = jnp.eye(dim, dtype=jnp.float32)
    table_rm = jax.lax.dot_general(
        embeddings, eye, (((1,), (0,)), ((), ())),
        preferred_element_type=jnp.float32,
        precision=jax.lax.Precision.HIGHEST)

    mesh = plsc.VectorSubcoreMesh(core_axis_name="c", subcore_axis_name="s")
    gather = functools.partial(
        pl.kernel,
        mesh=mesh,
        out_type=jax.ShapeDtypeStruct((total, dim), jnp.float32),
        scratch_types=[
            pltpu.VMEM((b_per_w,), jnp.int32),
            pltpu.VMEM((NBUF, CHUNK, dim), jnp.float32),
            pltpu.SemaphoreType.DMA((NBUF,)),
            pltpu.SemaphoreType.DMA((NBUF,)),
        ],
        compiler_params=pltpu.CompilerParams(use_tc_tiling_on_sc=False),
    )(functools.partial(_emb_gather, b_per_w=b_per_w, n_chunks=n_chunks))

    out = gather(idx_flat, table_rm)
    return out.reshape(batch, fields, dim)


# speedup vs baseline: 1.3392x; 1.3392x over previous
import functools

import jax
import jax.numpy as jnp
from jax import lax
from jax.experimental import pallas as pl
from jax.experimental.pallas import tpu as pltpu
from jax.experimental.pallas import tpu_sc as plsc

NUM_WORKERS = 32
CHUNK = 832
NBUF = 4


def _emb_gather(idx_hbm, table_hbm, out_hbm, idx_v, rows_v, gsems, wsems,
                *, b_per_w, n_chunks):
    wid = lax.axis_index("s") * 2 + lax.axis_index("c")
    base = wid * b_per_w

    pltpu.sync_copy(idx_hbm.at[pl.ds(base, b_per_w)], idx_v)

    def gather_start(ch, b):
        pltpu.async_copy(
            table_hbm.at[idx_v.at[pl.ds(ch * CHUNK, CHUNK)]],
            rows_v.at[b], gsems.at[b])

    def gather_wait(ch, b):
        pltpu.make_async_copy(
            table_hbm.at[idx_v.at[pl.ds(ch * CHUNK, CHUNK)]],
            rows_v.at[b], gsems.at[b]).wait()

    def writeback_start(ch, b):
        pltpu.async_copy(
            rows_v.at[b], out_hbm.at[pl.ds(base + ch * CHUNK, CHUNK)],
            wsems.at[b])

    def writeback_wait(ch, b):
        pltpu.make_async_copy(
            rows_v.at[b], out_hbm.at[pl.ds(base + ch * CHUNK, CHUNK)],
            wsems.at[b]).wait()

    for b in range(min(NBUF, n_chunks)):
        gather_start(b, b)
    for ch in range(n_chunks):
        b = ch % NBUF
        gather_wait(ch, b)
        writeback_start(ch, b)
        if ch + NBUF < n_chunks:
            writeback_wait(ch, b)
            gather_start(ch + NBUF, b)
    for ch in range(max(0, n_chunks - NBUF), n_chunks):
        writeback_wait(ch, ch % NBUF)


def kernel(token_ids, embeddings):
    batch, fields = token_ids.shape
    num_rows, dim = embeddings.shape
    total = batch * fields
    b_per_w = total // NUM_WORKERS
    n_chunks = b_per_w // CHUNK

    idx_flat = token_ids.reshape(total).astype(jnp.int32)

    mesh = plsc.VectorSubcoreMesh(core_axis_name="c", subcore_axis_name="s")
    gather = functools.partial(
        pl.kernel,
        mesh=mesh,
        out_type=jax.ShapeDtypeStruct((total, dim), jnp.float32),
        scratch_types=[
            pltpu.VMEM((b_per_w,), jnp.int32),
            pltpu.VMEM((NBUF, CHUNK, dim), jnp.float32),
            pltpu.SemaphoreType.DMA((NBUF,)),
            pltpu.SemaphoreType.DMA((NBUF,)),
        ],
        compiler_params=pltpu.CompilerParams(use_tc_tiling_on_sc=False),
    )(functools.partial(_emb_gather, b_per_w=b_per_w, n_chunks=n_chunks))

    out = gather(idx_flat, embeddings)
    return out.reshape(batch, fields, dim)
